# baseline (speedup 1.0000x reference)
import functools

import jax
import jax.numpy as jnp
from jax import lax
from jax.experimental import pallas as pl
from jax.experimental.pallas import tpu as pltpu
from jax.experimental.pallas import tpu_sc as plsc

N_NODES = 10000
N_EDGES = 320000
D_FEAT = 128
D_EDGE = 16
D_HIDDEN = 512

NC = 2
NS = 16
NW = NC * NS
EPW = N_EDGES // NW
CH = 80
NCH = EPW // CH
N_PAD = 10240
NPT = N_PAD // NS

_sc_mesh = plsc.VectorSubcoreMesh(core_axis_name="c", subcore_axis_name="s")


def _gather_body(x_hbm, dst_hbm, src_hbm, gobj_hbm, gsub_hbm,
                 idx_d, idx_s, buf_d, buf_s, sem_d, sem_s):
    cid = lax.axis_index("c")
    sid = lax.axis_index("s")
    wid = sid * NC + cid
    pltpu.sync_copy(dst_hbm.at[wid], idx_d)
    pltpu.sync_copy(src_hbm.at[wid], idx_s)

    def body(j, carry):
        cp_d = pltpu.async_copy(x_hbm.at[idx_d.at[j]], buf_d, sem_d)
        cp_s = pltpu.async_copy(x_hbm.at[idx_s.at[j]], buf_s, sem_s)
        cp_d.wait()
        pltpu.sync_copy(buf_d, gobj_hbm.at[wid, j])
        cp_s.wait()
        pltpu.sync_copy(buf_s, gsub_hbm.at[wid, j])
        return carry

    lax.fori_loop(0, NCH, body, 0)


D_PACK = D_FEAT // 2


@jax.jit
def _sc_gather(xp, dst3, src3):
    out_t = jax.ShapeDtypeStruct((NW, NCH, CH, D_PACK), jnp.float32)
    return pl.kernel(
        _gather_body,
        out_type=(out_t, out_t),
        mesh=_sc_mesh,
        compiler_params=pltpu.CompilerParams(use_tc_tiling_on_sc=False),
        scratch_types=[
            pltpu.VMEM((NCH, CH), jnp.int32),
            pltpu.VMEM((NCH, CH), jnp.int32),
            pltpu.VMEM((CH, D_PACK), jnp.float32),
            pltpu.VMEM((CH, D_PACK), jnp.float32),
            pltpu.SemaphoreType.DMA,
            pltpu.SemaphoreType.DMA,
        ],
    )(xp, dst3, src3)


def _scatter_body(msg_hbm, dst_hbm, zero_hbm, out_hbm,
                  idx_d, buf, acc, sem):
    cid = lax.axis_index("c")
    sid = lax.axis_index("s")
    wid = sid * NC + cid
    pltpu.sync_copy(zero_hbm.at[pl.ds(sid * NPT, NPT)],
                    acc.at[pl.ds(sid * NPT, NPT)])
    pltpu.sync_copy(dst_hbm.at[wid], idx_d)
    plsc.subcore_barrier()

    def body(j, carry):
        pltpu.sync_copy(msg_hbm.at[wid, j], buf)
        pltpu.sync_copy(buf, acc.at[idx_d.at[j]], add=True)
        return carry

    lax.fori_loop(0, NCH, body, 0)
    plsc.subcore_barrier()
    pltpu.sync_copy(acc.at[pl.ds(sid * NPT, NPT)],
                    out_hbm.at[cid, pl.ds(sid * NPT, NPT)])


@jax.jit
def _sc_scatter(msg4, dst3, zeros_nm):
    return pl.kernel(
        _scatter_body,
        out_type=jax.ShapeDtypeStruct((NC, N_PAD, D_FEAT), jnp.float32),
        mesh=_sc_mesh,
        scratch_types=[
            pltpu.VMEM((NCH, CH), jnp.int32),
            pltpu.VMEM((CH, D_FEAT), jnp.float32),
            pltpu.VMEM_SHARED((N_PAD, D_FEAT), jnp.float32),
            pltpu.SemaphoreType.DMA,
        ],
    )(msg4, dst3, zeros_nm)


def _count_body(dst_hbm, zero_hbm, ones_hbm, out_hbm, idx_d, buf, acc, sem):
    cid = lax.axis_index("c")
    sid = lax.axis_index("s")
    wid = sid * NC + cid
    pltpu.sync_copy(zero_hbm.at[pl.ds(sid * NPT, NPT)],
                    acc.at[pl.ds(sid * NPT, NPT)])
    pltpu.sync_copy(dst_hbm.at[wid], idx_d)
    pltpu.sync_copy(ones_hbm, buf)
    plsc.subcore_barrier()

    def body(j, carry):
        pltpu.sync_copy(buf, acc.at[idx_d.at[j]], add=True)
        return carry

    lax.fori_loop(0, NCH, body, 0)
    plsc.subcore_barrier()
    pltpu.sync_copy(acc.at[pl.ds(sid * NPT, NPT)],
                    out_hbm.at[cid, pl.ds(sid * NPT, NPT)])


@jax.jit
def _sc_count(dst3, zeros_nm, ones_ch):
    return pl.kernel(
        _count_body,
        out_type=jax.ShapeDtypeStruct((NC, N_PAD, D_FEAT), jnp.float32),
        mesh=_sc_mesh,
        scratch_types=[
            pltpu.VMEM((NCH, CH), jnp.int32),
            pltpu.VMEM((CH, D_FEAT), jnp.float32),
            pltpu.VMEM_SHARED((N_PAD, D_FEAT), jnp.float32),
            pltpu.SemaphoreType.DMA,
        ],
    )(dst3, zeros_nm, ones_ch)


BE = 512
assert N_EDGES % BE == 0


def _mlp_body(gobj, gsub, ea, w1a, w1e, w1b, b1, w2, b2, out):
    h = jnp.dot(gobj[...], w1a[...], preferred_element_type=jnp.float32)
    h = h + jnp.dot(ea[...], w1e[...], preferred_element_type=jnp.float32)
    h = h + jnp.dot(gsub[...], w1b[...], preferred_element_type=jnp.float32)
    h = jnp.maximum(h + b1[...], 0.0).astype(jnp.bfloat16)
    out[...] = jnp.dot(h, w2[...], preferred_element_type=jnp.float32) + b2[...]


@jax.jit
def _tc_mlp(gobj, gsub, ea, w1a, w1e, w1b, b1, w2, b2):
    grid = (N_EDGES // BE,)
    full = lambda shape: pl.BlockSpec(shape, lambda i: (0, 0))
    return pl.pallas_call(
        _mlp_body,
        grid=grid,
        in_specs=[
            pl.BlockSpec((BE, D_FEAT), lambda i: (i, 0)),
            pl.BlockSpec((BE, D_FEAT), lambda i: (i, 0)),
            pl.BlockSpec((BE, D_EDGE), lambda i: (i, 0)),
            full((D_FEAT, D_HIDDEN)),
            full((D_EDGE, D_HIDDEN)),
            full((D_FEAT, D_HIDDEN)),
            full((1, D_HIDDEN)),
            full((D_HIDDEN, D_FEAT)),
            full((1, D_FEAT)),
        ],
        out_specs=pl.BlockSpec((BE, D_FEAT), lambda i: (i, 0)),
        out_shape=jax.ShapeDtypeStruct((N_EDGES, D_FEAT), jnp.float32),
    )(gobj, gsub, ea, w1a, w1e, w1b, b1, w2, b2)


def _combine_body(s_ref, c_ref, out_ref):
    s = s_ref[0, :N_NODES, :] + s_ref[1, :N_NODES, :]
    c = jnp.maximum(c_ref[0, :N_NODES, :] + c_ref[1, :N_NODES, :], 1.0)
    out_ref[...] = s / c


@jax.jit
def _tc_combine(partials, counts):
    return pl.pallas_call(
        _combine_body,
        out_shape=jax.ShapeDtypeStruct((N_NODES, D_FEAT), jnp.float32),
    )(partials, counts)


def kernel(x, edge_index, edge_attr, W1, b1, W2, b2):
    dst = edge_index[1].astype(jnp.int32)
    src = edge_index[0].astype(jnp.int32)
    dst3 = dst.reshape(NW, NCH, CH)
    src3 = src.reshape(NW, NCH, CH)

    xb = x.astype(jnp.bfloat16)
    xp = lax.bitcast_convert_type(xb.reshape(N_NODES, D_PACK, 2),
                                  jnp.float32)
    gobj4, gsub4 = _sc_gather(xp, dst3, src3)
    unpack = lambda g: lax.bitcast_convert_type(
        g.reshape(N_EDGES, D_PACK), jnp.bfloat16).reshape(N_EDGES, D_FEAT)
    gobj = unpack(gobj4)
    gsub = unpack(gsub4)

    w1a = W1[:D_FEAT].astype(jnp.bfloat16)
    w1e = W1[D_FEAT:D_FEAT + D_EDGE].astype(jnp.bfloat16)
    w1b = W1[D_FEAT + D_EDGE:].astype(jnp.bfloat16)
    msg = _tc_mlp(gobj, gsub, edge_attr.astype(jnp.bfloat16),
                  w1a, w1e, w1b, b1.reshape(1, D_HIDDEN),
                  W2.astype(jnp.bfloat16), b2.reshape(1, D_FEAT))

    zeros_nm = jnp.zeros((N_PAD, D_FEAT), jnp.float32)
    ones_ch = jnp.ones((CH, D_FEAT), jnp.float32)
    partials = _sc_scatter(msg.reshape(NW, NCH, CH, D_FEAT), dst3, zeros_nm)
    counts = _sc_count(dst3, zeros_nm, ones_ch)
    return _tc_combine(partials, counts)


# speedup vs baseline: 2.2040x; 2.2040x over previous
import functools

import jax
import jax.numpy as jnp
from jax import lax
from jax.experimental import pallas as pl
from jax.experimental.pallas import tpu as pltpu
from jax.experimental.pallas import tpu_sc as plsc

N_NODES = 10000
N_EDGES = 320000
D_FEAT = 128
D_EDGE = 16
D_HIDDEN = 512

NC = 2
NS = 16
NW = NC * NS
EPW = N_EDGES // NW
CH = 80
NCH = EPW // CH
N_PAD = 10240
NPT = N_PAD // NS

_sc_mesh = plsc.VectorSubcoreMesh(core_axis_name="c", subcore_axis_name="s")


def _gather_body(x_hbm, dst_hbm, src_hbm, gobj_hbm, gsub_hbm,
                 idx_d, idx_s, buf_d, buf_s, sem_d, sem_s):
    cid = lax.axis_index("c")
    sid = lax.axis_index("s")
    wid = sid * NC + cid
    pltpu.sync_copy(dst_hbm.at[wid], idx_d)
    pltpu.sync_copy(src_hbm.at[wid], idx_s)

    def body(j, carry):
        cp_d = pltpu.async_copy(x_hbm.at[idx_d.at[j]], buf_d, sem_d)
        cp_s = pltpu.async_copy(x_hbm.at[idx_s.at[j]], buf_s, sem_s)
        cp_d.wait()
        pltpu.sync_copy(buf_d, gobj_hbm.at[wid, j])
        cp_s.wait()
        pltpu.sync_copy(buf_s, gsub_hbm.at[wid, j])
        return carry

    lax.fori_loop(0, NCH, body, 0)


@jax.jit
def _sc_gather(xp, dst3, src3):
    out_t = jax.ShapeDtypeStruct((NW, NCH, CH, D_FEAT), jnp.float32)
    return pl.kernel(
        _gather_body,
        out_type=(out_t, out_t),
        mesh=_sc_mesh,
        scratch_types=[
            pltpu.VMEM((NCH, CH), jnp.int32),
            pltpu.VMEM((NCH, CH), jnp.int32),
            pltpu.VMEM((CH, D_FEAT), jnp.float32),
            pltpu.VMEM((CH, D_FEAT), jnp.float32),
            pltpu.SemaphoreType.DMA,
            pltpu.SemaphoreType.DMA,
        ],
    )(xp, dst3, src3)


def _scatter_body(msg_hbm, dst_hbm, zero_hbm, out_hbm,
                  idx_d, buf, acc, sem):
    cid = lax.axis_index("c")
    sid = lax.axis_index("s")
    wid = sid * NC + cid
    pltpu.sync_copy(zero_hbm.at[pl.ds(sid * NPT, NPT)],
                    acc.at[pl.ds(sid * NPT, NPT)])
    pltpu.sync_copy(dst_hbm.at[wid], idx_d)
    plsc.subcore_barrier()

    def body(j, carry):
        pltpu.sync_copy(msg_hbm.at[wid, j], buf)
        pltpu.sync_copy(buf, acc.at[idx_d.at[j]], add=True)
        return carry

    lax.fori_loop(0, NCH, body, 0)
    plsc.subcore_barrier()
    pltpu.sync_copy(acc.at[pl.ds(sid * NPT, NPT)],
                    out_hbm.at[cid, pl.ds(sid * NPT, NPT)])


@jax.jit
def _sc_scatter(msg4, dst3, zeros_nm):
    return pl.kernel(
        _scatter_body,
        out_type=jax.ShapeDtypeStruct((NC, N_PAD, D_FEAT), jnp.float32),
        mesh=_sc_mesh,
        scratch_types=[
            pltpu.VMEM((NCH, CH), jnp.int32),
            pltpu.VMEM((CH, D_FEAT), jnp.float32),
            pltpu.VMEM_SHARED((N_PAD, D_FEAT), jnp.float32),
            pltpu.SemaphoreType.DMA,
        ],
    )(msg4, dst3, zeros_nm)


def _count_body(dst_hbm, zero_hbm, ones_hbm, out_hbm, idx_d, buf, acc, sem):
    cid = lax.axis_index("c")
    sid = lax.axis_index("s")
    wid = sid * NC + cid
    pltpu.sync_copy(zero_hbm.at[pl.ds(sid * NPT, NPT)],
                    acc.at[pl.ds(sid * NPT, NPT)])
    pltpu.sync_copy(dst_hbm.at[wid], idx_d)
    pltpu.sync_copy(ones_hbm, buf)
    plsc.subcore_barrier()

    def body(j, carry):
        pltpu.sync_copy(buf, acc.at[idx_d.at[j]], add=True)
        return carry

    lax.fori_loop(0, NCH, body, 0)
    plsc.subcore_barrier()
    pltpu.sync_copy(acc.at[pl.ds(sid * NPT, NPT)],
                    out_hbm.at[cid, pl.ds(sid * NPT, NPT)])


@jax.jit
def _sc_count(dst3, zeros_nm, ones_ch):
    return pl.kernel(
        _count_body,
        out_type=jax.ShapeDtypeStruct((NC, N_PAD, D_FEAT), jnp.float32),
        mesh=_sc_mesh,
        scratch_types=[
            pltpu.VMEM((NCH, CH), jnp.int32),
            pltpu.VMEM((CH, D_FEAT), jnp.float32),
            pltpu.VMEM_SHARED((N_PAD, D_FEAT), jnp.float32),
            pltpu.SemaphoreType.DMA,
        ],
    )(dst3, zeros_nm, ones_ch)


BE = 512
assert N_EDGES % BE == 0


def _mlp_body(gobj, gsub, ea, w1a, w1e, w1b, b1, w2, b2, out):
    go = gobj[...].astype(jnp.bfloat16)
    gs = gsub[...].astype(jnp.bfloat16)
    e = ea[...].astype(jnp.bfloat16)
    h = jnp.dot(go, w1a[...], preferred_element_type=jnp.float32)
    h = h + jnp.dot(e, w1e[...], preferred_element_type=jnp.float32)
    h = h + jnp.dot(gs, w1b[...], preferred_element_type=jnp.float32)
    h = jnp.maximum(h + b1[...], 0.0).astype(jnp.bfloat16)
    out[...] = jnp.dot(h, w2[...], preferred_element_type=jnp.float32) + b2[...]


@jax.jit
def _tc_mlp(gobj, gsub, ea, w1a, w1e, w1b, b1, w2, b2):
    grid = (N_EDGES // BE,)
    full = lambda shape: pl.BlockSpec(shape, lambda i: (0, 0))
    return pl.pallas_call(
        _mlp_body,
        grid=grid,
        in_specs=[
            pl.BlockSpec((BE, D_FEAT), lambda i: (i, 0)),
            pl.BlockSpec((BE, D_FEAT), lambda i: (i, 0)),
            pl.BlockSpec((BE, D_EDGE), lambda i: (i, 0)),
            full((D_FEAT, D_HIDDEN)),
            full((D_EDGE, D_HIDDEN)),
            full((D_FEAT, D_HIDDEN)),
            full((1, D_HIDDEN)),
            full((D_HIDDEN, D_FEAT)),
            full((1, D_FEAT)),
        ],
        out_specs=pl.BlockSpec((BE, D_FEAT), lambda i: (i, 0)),
        out_shape=jax.ShapeDtypeStruct((N_EDGES, D_FEAT), jnp.float32),
    )(gobj, gsub, ea, w1a, w1e, w1b, b1, w2, b2)


def _combine_body(s_ref, c_ref, out_ref):
    s = s_ref[0, :N_NODES, :] + s_ref[1, :N_NODES, :]
    c = jnp.maximum(c_ref[0, :N_NODES, :] + c_ref[1, :N_NODES, :], 1.0)
    out_ref[...] = s / c


@jax.jit
def _tc_combine(partials, counts):
    return pl.pallas_call(
        _combine_body,
        out_shape=jax.ShapeDtypeStruct((N_NODES, D_FEAT), jnp.float32),
    )(partials, counts)


def kernel(x, edge_index, edge_attr, W1, b1, W2, b2):
    dst = edge_index[1].astype(jnp.int32)
    src = edge_index[0].astype(jnp.int32)
    dst3 = dst.reshape(NW, NCH, CH)
    src3 = src.reshape(NW, NCH, CH)

    gobj4, gsub4 = _sc_gather(x, dst3, src3)
    gobj = gobj4.reshape(N_EDGES, D_FEAT)
    gsub = gsub4.reshape(N_EDGES, D_FEAT)

    w1a = W1[:D_FEAT].astype(jnp.bfloat16)
    w1e = W1[D_FEAT:D_FEAT + D_EDGE].astype(jnp.bfloat16)
    w1b = W1[D_FEAT + D_EDGE:].astype(jnp.bfloat16)
    msg = _tc_mlp(gobj, gsub, edge_attr,
                  w1a, w1e, w1b, b1.reshape(1, D_HIDDEN),
                  W2.astype(jnp.bfloat16), b2.reshape(1, D_FEAT))

    zeros_nm = jnp.zeros((N_PAD, D_FEAT), jnp.float32)
    ones_ch = jnp.ones((CH, D_FEAT), jnp.float32)
    partials = _sc_scatter(msg.reshape(NW, NCH, CH, D_FEAT), dst3, zeros_nm)
    counts = _sc_count(dst3, zeros_nm, ones_ch)
    return _tc_combine(partials, counts)


# speedup vs baseline: 3.0306x; 1.3751x over previous
import functools

import jax
import jax.numpy as jnp
from jax import lax
from jax.experimental import pallas as pl
from jax.experimental.pallas import tpu as pltpu
from jax.experimental.pallas import tpu_sc as plsc

N_NODES = 10000
N_EDGES = 320000
D_FEAT = 128
D_PAIR = 2 * D_FEAT
D_EDGE = 16
D_HIDDEN = 512

NC = 2
NS = 16
NW = NC * NS
EPW = N_EDGES // NW
CH = 80
NCH = EPW // CH
NCH2 = (NCH - 1) // 2
N_PAD = 10240
NPT = N_PAD // NS

_sc_mesh = plsc.VectorSubcoreMesh(core_axis_name="c", subcore_axis_name="s")


def _gather_body(x_hbm, dst_hbm, src_hbm, g_hbm, idx_d, idx_s,
                 bA_d, bA_s, bB_d, bB_s,
                 s_gAd, s_gAs, s_gBd, s_gBs, s_wAd, s_wAs, s_wBd, s_wBs):
    cid = lax.axis_index("c")
    sid = lax.axis_index("s")
    wid = sid * NC + cid
    pltpu.sync_copy(dst_hbm.at[wid], idx_d)
    pltpu.sync_copy(src_hbm.at[wid], idx_s)

    obj_lanes = pl.ds(0, D_FEAT)
    sub_lanes = pl.ds(D_FEAT, D_FEAT)

    pltpu.async_copy(x_hbm.at[idx_d.at[0]], bA_d, s_gAd)
    pltpu.async_copy(x_hbm.at[idx_s.at[0]], bA_s, s_gAs)

    def body(jj, carry):
        j0 = 2 * jj
        j1 = j0 + 1
        j2 = j0 + 2
        pltpu.make_async_copy(x_hbm.at[idx_d.at[j0]], bA_d, s_gAd).wait()
        pltpu.make_async_copy(x_hbm.at[idx_s.at[j0]], bA_s, s_gAs).wait()
        pltpu.async_copy(x_hbm.at[idx_d.at[j1]], bB_d, s_gBd)
        pltpu.async_copy(x_hbm.at[idx_s.at[j1]], bB_s, s_gBs)
        wAd = pltpu.async_copy(bA_d, g_hbm.at[wid, j0, :, obj_lanes], s_wAd)
        wAs = pltpu.async_copy(bA_s, g_hbm.at[wid, j0, :, sub_lanes], s_wAs)
        pltpu.make_async_copy(x_hbm.at[idx_d.at[j1]], bB_d, s_gBd).wait()
        pltpu.make_async_copy(x_hbm.at[idx_s.at[j1]], bB_s, s_gBs).wait()
        wBd = pltpu.async_copy(bB_d, g_hbm.at[wid, j1, :, obj_lanes], s_wBd)
        wBs = pltpu.async_copy(bB_s, g_hbm.at[wid, j1, :, sub_lanes], s_wBs)
        wAd.wait()
        wAs.wait()
        pltpu.async_copy(x_hbm.at[idx_d.at[j2]], bA_d, s_gAd)
        pltpu.async_copy(x_hbm.at[idx_s.at[j2]], bA_s, s_gAs)
        wBd.wait()
        wBs.wait()
        return carry

    lax.fori_loop(0, NCH2, body, 0)
    j = NCH - 1
    pltpu.make_async_copy(x_hbm.at[idx_d.at[j]], bA_d, s_gAd).wait()
    pltpu.make_async_copy(x_hbm.at[idx_s.at[j]], bA_s, s_gAs).wait()
    pltpu.sync_copy(bA_d, g_hbm.at[wid, j, :, obj_lanes])
    pltpu.sync_copy(bA_s, g_hbm.at[wid, j, :, sub_lanes])


@jax.jit
def _sc_gather(x, dst3, src3):
    buf = lambda: pltpu.VMEM((CH, D_FEAT), jnp.float32)
    return pl.kernel(
        _gather_body,
        out_type=jax.ShapeDtypeStruct((NW, NCH, CH, D_PAIR), jnp.float32),
        mesh=_sc_mesh,
        scratch_types=[
            pltpu.VMEM((NCH, CH), jnp.int32),
            pltpu.VMEM((NCH, CH), jnp.int32),
            buf(), buf(), buf(), buf(),
        ] + [pltpu.SemaphoreType.DMA] * 8,
    )(x, dst3, src3)


def _scatter_body(msg_hbm, dst_hbm, zero_hbm, out_hbm,
                  idx_d, bA, bB, acc, s_lA, s_lB, s_sA, s_sB):
    cid = lax.axis_index("c")
    sid = lax.axis_index("s")
    wid = sid * NC + cid
    pltpu.sync_copy(zero_hbm.at[pl.ds(sid * NPT, NPT)],
                    acc.at[pl.ds(sid * NPT, NPT)])
    pltpu.sync_copy(dst_hbm.at[wid], idx_d)
    plsc.subcore_barrier()

    pltpu.async_copy(msg_hbm.at[wid, 0], bA, s_lA)

    def body(jj, carry):
        j0 = 2 * jj
        j1 = j0 + 1
        j2 = j0 + 2
        pltpu.make_async_copy(msg_hbm.at[wid, j0], bA, s_lA).wait()
        pltpu.async_copy(msg_hbm.at[wid, j1], bB, s_lB)
        sA = pltpu.async_copy(bA, acc.at[idx_d.at[j0]], s_sA, add=True)
        pltpu.make_async_copy(msg_hbm.at[wid, j1], bB, s_lB).wait()
        sB = pltpu.async_copy(bB, acc.at[idx_d.at[j1]], s_sB, add=True)
        sA.wait()
        pltpu.async_copy(msg_hbm.at[wid, j2], bA, s_lA)
        sB.wait()
        return carry

    lax.fori_loop(0, NCH2, body, 0)
    j = NCH - 1
    pltpu.make_async_copy(msg_hbm.at[wid, j], bA, s_lA).wait()
    pltpu.sync_copy(bA, acc.at[idx_d.at[j]], add=True)
    plsc.subcore_barrier()
    pltpu.sync_copy(acc.at[pl.ds(sid * NPT, NPT)],
                    out_hbm.at[cid, pl.ds(sid * NPT, NPT)])


@jax.jit
def _sc_scatter(msg4, dst3, zeros_nm):
    return pl.kernel(
        _scatter_body,
        out_type=jax.ShapeDtypeStruct((NC, N_PAD, D_FEAT), jnp.float32),
        mesh=_sc_mesh,
        scratch_types=[
            pltpu.VMEM((NCH, CH), jnp.int32),
            pltpu.VMEM((CH, D_FEAT), jnp.float32),
            pltpu.VMEM((CH, D_FEAT), jnp.float32),
            pltpu.VMEM_SHARED((N_PAD, D_FEAT), jnp.float32),
        ] + [pltpu.SemaphoreType.DMA] * 4,
    )(msg4, dst3, zeros_nm)


def _count_body(dst_hbm, zero_hbm, ones_hbm, out_hbm, idx_d, buf, acc, sem):
    cid = lax.axis_index("c")
    sid = lax.axis_index("s")
    wid = sid * NC + cid
    pltpu.sync_copy(zero_hbm.at[pl.ds(sid * NPT, NPT)],
                    acc.at[pl.ds(sid * NPT, NPT)])
    pltpu.sync_copy(dst_hbm.at[wid], idx_d)
    pltpu.sync_copy(ones_hbm, buf)
    plsc.subcore_barrier()

    def body(j, carry):
        pltpu.sync_copy(buf, acc.at[idx_d.at[j]], add=True)
        return carry

    lax.fori_loop(0, NCH, body, 0)
    plsc.subcore_barrier()
    pltpu.sync_copy(acc.at[pl.ds(sid * NPT, NPT)],
                    out_hbm.at[cid, pl.ds(sid * NPT, NPT)])


@jax.jit
def _sc_count(dst3, zeros_nm, ones_ch):
    return pl.kernel(
        _count_body,
        out_type=jax.ShapeDtypeStruct((NC, N_PAD, D_FEAT), jnp.float32),
        mesh=_sc_mesh,
        scratch_types=[
            pltpu.VMEM((NCH, CH), jnp.int32),
            pltpu.VMEM((CH, D_FEAT), jnp.float32),
            pltpu.VMEM_SHARED((N_PAD, D_FEAT), jnp.float32),
            pltpu.SemaphoreType.DMA,
        ],
    )(dst3, zeros_nm, ones_ch)


BE = 1280
assert N_EDGES % BE == 0


def _mlp_body(g, ea, w1os, w1e, b1, w2, b2, out):
    gb = g[...].astype(jnp.bfloat16)
    e = ea[...].astype(jnp.bfloat16)
    h = jnp.dot(gb, w1os[...], preferred_element_type=jnp.float32)
    h = h + jnp.dot(e, w1e[...], preferred_element_type=jnp.float32)
    h = jnp.maximum(h + b1[...], 0.0).astype(jnp.bfloat16)
    out[...] = jnp.dot(h, w2[...], preferred_element_type=jnp.float32) + b2[...]


@jax.jit
def _tc_mlp(g, ea, w1os, w1e, b1, w2, b2):
    grid = (N_EDGES // BE,)
    full = lambda shape: pl.BlockSpec(shape, lambda i: (0, 0))
    return pl.pallas_call(
        _mlp_body,
        grid=grid,
        in_specs=[
            pl.BlockSpec((BE, D_PAIR), lambda i: (i, 0)),
            pl.BlockSpec((BE, D_EDGE), lambda i: (i, 0)),
            full((D_PAIR, D_HIDDEN)),
            full((D_EDGE, D_HIDDEN)),
            full((1, D_HIDDEN)),
            full((D_HIDDEN, D_FEAT)),
            full((1, D_FEAT)),
        ],
        out_specs=pl.BlockSpec((BE, D_FEAT), lambda i: (i, 0)),
        out_shape=jax.ShapeDtypeStruct((N_EDGES, D_FEAT), jnp.float32),
    )(g, ea, w1os, w1e, b1, w2, b2)


def _combine_body(s_ref, c_ref, out_ref):
    s = s_ref[0, :N_NODES, :] + s_ref[1, :N_NODES, :]
    c = jnp.maximum(c_ref[0, :N_NODES, :] + c_ref[1, :N_NODES, :], 1.0)
    out_ref[...] = s / c


@jax.jit
def _tc_combine(partials, counts):
    return pl.pallas_call(
        _combine_body,
        out_shape=jax.ShapeDtypeStruct((N_NODES, D_FEAT), jnp.float32),
    )(partials, counts)


def kernel(x, edge_index, edge_attr, W1, b1, W2, b2):
    dst = edge_index[1].astype(jnp.int32)
    src = edge_index[0].astype(jnp.int32)
    dst3 = dst.reshape(NW, NCH, CH)
    src3 = src.reshape(NW, NCH, CH)

    g4 = _sc_gather(x, dst3, src3)
    g = g4.reshape(N_EDGES, D_PAIR)

    w1os = jnp.concatenate(
        [W1[:D_FEAT], W1[D_FEAT + D_EDGE:]], axis=0).astype(jnp.bfloat16)
    w1e = W1[D_FEAT:D_FEAT + D_EDGE].astype(jnp.bfloat16)
    msg = _tc_mlp(g, edge_attr, w1os, w1e, b1.reshape(1, D_HIDDEN),
                  W2.astype(jnp.bfloat16), b2.reshape(1, D_FEAT))

    zeros_nm = jnp.zeros((N_PAD, D_FEAT), jnp.float32)
    ones_ch = jnp.ones((CH, D_FEAT), jnp.float32)
    partials = _sc_scatter(msg.reshape(NW, NCH, CH, D_FEAT), dst3, zeros_nm)
    counts = _sc_count(dst3, zeros_nm, ones_ch)
    return _tc_combine(partials, counts)


# speedup vs baseline: 3.8042x; 1.2552x over previous
import functools

import jax
import jax.numpy as jnp
from jax import lax
from jax.experimental import pallas as pl
from jax.experimental.pallas import tpu as pltpu
from jax.experimental.pallas import tpu_sc as plsc

N_NODES = 10000
N_EDGES = 320000
D_FEAT = 128
D_PAIR = 2 * D_FEAT
D_EDGE = 16
D_HIDDEN = 512

NC = 2
NS = 16
NW = NC * NS
NG = 5
EG = N_EDGES // NG
EPWG = EG // NW
CH = 80
NCHG = EPWG // CH
NCH2G = (NCHG - 1) // 2
NCHF = N_EDGES // NW // CH
N_PAD = 10240
NPT = N_PAD // NS

_sc_mesh = plsc.VectorSubcoreMesh(core_axis_name="c", subcore_axis_name="s")


def _gather_body(x_hbm, dst_hbm, src_hbm, g_hbm, idx_d, idx_s,
                 bA_d, bA_s, bB_d, bB_s,
                 s_gAd, s_gAs, s_gBd, s_gBs, s_wAd, s_wAs, s_wBd, s_wBs):
    cid = lax.axis_index("c")
    sid = lax.axis_index("s")
    wid = sid * NC + cid
    pltpu.sync_copy(dst_hbm.at[wid], idx_d)
    pltpu.sync_copy(src_hbm.at[wid], idx_s)

    obj_lanes = pl.ds(0, D_FEAT)
    sub_lanes = pl.ds(D_FEAT, D_FEAT)

    pltpu.async_copy(x_hbm.at[idx_d.at[0]], bA_d, s_gAd)
    pltpu.async_copy(x_hbm.at[idx_s.at[0]], bA_s, s_gAs)

    def body(jj, carry):
        j0 = 2 * jj
        j1 = j0 + 1
        j2 = j0 + 2
        pltpu.make_async_copy(x_hbm.at[idx_d.at[j0]], bA_d, s_gAd).wait()
        pltpu.make_async_copy(x_hbm.at[idx_s.at[j0]], bA_s, s_gAs).wait()
        pltpu.async_copy(x_hbm.at[idx_d.at[j1]], bB_d, s_gBd)
        pltpu.async_copy(x_hbm.at[idx_s.at[j1]], bB_s, s_gBs)
        wAd = pltpu.async_copy(bA_d, g_hbm.at[wid, j0, :, obj_lanes], s_wAd)
        wAs = pltpu.async_copy(bA_s, g_hbm.at[wid, j0, :, sub_lanes], s_wAs)
        pltpu.make_async_copy(x_hbm.at[idx_d.at[j1]], bB_d, s_gBd).wait()
        pltpu.make_async_copy(x_hbm.at[idx_s.at[j1]], bB_s, s_gBs).wait()
        wBd = pltpu.async_copy(bB_d, g_hbm.at[wid, j1, :, obj_lanes], s_wBd)
        wBs = pltpu.async_copy(bB_s, g_hbm.at[wid, j1, :, sub_lanes], s_wBs)
        wAd.wait()
        wAs.wait()
        pltpu.async_copy(x_hbm.at[idx_d.at[j2]], bA_d, s_gAd)
        pltpu.async_copy(x_hbm.at[idx_s.at[j2]], bA_s, s_gAs)
        wBd.wait()
        wBs.wait()
        return carry

    lax.fori_loop(0, NCH2G, body, 0)
    j = NCHG - 1
    pltpu.make_async_copy(x_hbm.at[idx_d.at[j]], bA_d, s_gAd).wait()
    pltpu.make_async_copy(x_hbm.at[idx_s.at[j]], bA_s, s_gAs).wait()
    pltpu.sync_copy(bA_d, g_hbm.at[wid, j, :, obj_lanes])
    pltpu.sync_copy(bA_s, g_hbm.at[wid, j, :, sub_lanes])


@jax.jit
def _sc_gather(x, dstg, srcg):
    buf = lambda: pltpu.VMEM((CH, D_FEAT), jnp.float32)
    return pl.kernel(
        _gather_body,
        out_type=jax.ShapeDtypeStruct((NW, NCHG, CH, D_PAIR), jnp.float32),
        mesh=_sc_mesh,
        scratch_types=[
            pltpu.VMEM((NCHG, CH), jnp.int32),
            pltpu.VMEM((NCHG, CH), jnp.int32),
            buf(), buf(), buf(), buf(),
        ] + [pltpu.SemaphoreType.DMA] * 8,
    )(x, dstg, srcg)


def _scatter_body(msg_hbm, dst_hbm, zero_hbm, out_hbm,
                  idx_d, bA, bB, acc, s_lA, s_lB, s_sA, s_sB):
    cid = lax.axis_index("c")
    sid = lax.axis_index("s")
    wid = sid * NC + cid
    pltpu.sync_copy(zero_hbm.at[pl.ds(sid * NPT, NPT)],
                    acc.at[pl.ds(sid * NPT, NPT)])
    pltpu.sync_copy(dst_hbm.at[wid], idx_d)
    plsc.subcore_barrier()

    pltpu.async_copy(msg_hbm.at[wid, 0], bA, s_lA)

    def body(jj, carry):
        j0 = 2 * jj
        j1 = j0 + 1
        j2 = j0 + 2
        pltpu.make_async_copy(msg_hbm.at[wid, j0], bA, s_lA).wait()
        pltpu.async_copy(msg_hbm.at[wid, j1], bB, s_lB)
        sA = pltpu.async_copy(bA, acc.at[idx_d.at[j0]], s_sA, add=True)
        pltpu.make_async_copy(msg_hbm.at[wid, j1], bB, s_lB).wait()
        sB = pltpu.async_copy(bB, acc.at[idx_d.at[j1]], s_sB, add=True)
        sA.wait()
        pltpu.async_copy(msg_hbm.at[wid, j2], bA, s_lA)
        sB.wait()
        return carry

    lax.fori_loop(0, NCH2G, body, 0)
    j = NCHG - 1
    pltpu.make_async_copy(msg_hbm.at[wid, j], bA, s_lA).wait()
    pltpu.sync_copy(bA, acc.at[idx_d.at[j]], add=True)
    plsc.subcore_barrier()
    pltpu.sync_copy(acc.at[pl.ds(sid * NPT, NPT)],
                    out_hbm.at[cid, pl.ds(sid * NPT, NPT)])


@jax.jit
def _sc_scatter(msg4, dstg, zeros_nm):
    return pl.kernel(
        _scatter_body,
        out_type=jax.ShapeDtypeStruct((NC, N_PAD, D_FEAT), jnp.float32),
        mesh=_sc_mesh,
        scratch_types=[
            pltpu.VMEM((NCHG, CH), jnp.int32),
            pltpu.VMEM((CH, D_FEAT), jnp.float32),
            pltpu.VMEM((CH, D_FEAT), jnp.float32),
            pltpu.VMEM_SHARED((N_PAD, D_FEAT), jnp.float32),
        ] + [pltpu.SemaphoreType.DMA] * 4,
    )(msg4, dstg, zeros_nm)


def _count_body(dst_hbm, zero_hbm, ones_hbm, out_hbm, idx_d, buf, acc, sem):
    cid = lax.axis_index("c")
    sid = lax.axis_index("s")
    wid = sid * NC + cid
    pltpu.sync_copy(zero_hbm.at[pl.ds(sid * NPT, NPT)],
                    acc.at[pl.ds(sid * NPT, NPT)])
    pltpu.sync_copy(dst_hbm.at[wid], idx_d)
    pltpu.sync_copy(ones_hbm, buf)
    plsc.subcore_barrier()

    def body(j, carry):
        pltpu.sync_copy(buf, acc.at[idx_d.at[j]], add=True)
        return carry

    lax.fori_loop(0, NCHF, body, 0)
    plsc.subcore_barrier()
    pltpu.sync_copy(acc.at[pl.ds(sid * NPT, NPT)],
                    out_hbm.at[cid, pl.ds(sid * NPT, NPT)])


@jax.jit
def _sc_count(dst3, zeros_nm, ones_ch):
    return pl.kernel(
        _count_body,
        out_type=jax.ShapeDtypeStruct((NC, N_PAD, D_FEAT), jnp.float32),
        mesh=_sc_mesh,
        scratch_types=[
            pltpu.VMEM((NCHF, CH), jnp.int32),
            pltpu.VMEM((CH, D_FEAT), jnp.float32),
            pltpu.VMEM_SHARED((N_PAD, D_FEAT), jnp.float32),
            pltpu.SemaphoreType.DMA,
        ],
    )(dst3, zeros_nm, ones_ch)


BE = 1280
assert EG % BE == 0


def _mlp_body(g, eat, w1os, w1e, b1, w2, b2, out):
    gb = g[...].astype(jnp.bfloat16)
    e = eat[...].astype(jnp.bfloat16)
    h = jnp.dot(gb, w1os[...], preferred_element_type=jnp.float32)
    h = h + lax.dot_general(e, w1e[...], (((0,), (0,)), ((), ())),
                            preferred_element_type=jnp.float32)
    h = jnp.maximum(h + b1[...], 0.0).astype(jnp.bfloat16)
    out[...] = jnp.dot(h, w2[...], preferred_element_type=jnp.float32) + b2[...]


@jax.jit
def _tc_mlp(g, eat, w1os, w1e, b1, w2, b2):
    grid = (EG // BE,)
    full = lambda shape: pl.BlockSpec(shape, lambda i: (0, 0))
    return pl.pallas_call(
        _mlp_body,
        grid=grid,
        in_specs=[
            pl.BlockSpec((BE, D_PAIR), lambda i: (i, 0)),
            pl.BlockSpec((D_EDGE, BE), lambda i: (0, i)),
            full((D_PAIR, D_HIDDEN)),
            full((D_EDGE, D_HIDDEN)),
            full((1, D_HIDDEN)),
            full((D_HIDDEN, D_FEAT)),
            full((1, D_FEAT)),
        ],
        out_specs=pl.BlockSpec((BE, D_FEAT), lambda i: (i, 0)),
        out_shape=jax.ShapeDtypeStruct((EG, D_FEAT), jnp.float32),
    )(g, eat, w1os, w1e, b1, w2, b2)


BN = 2000


def _combine_body(p0, p1, p2, p3, p4, c, out):
    s = p0[0] + p0[1]
    for p in (p1, p2, p3, p4):
        s = s + p[0] + p[1]
    cc = jnp.maximum(c[0] + c[1], 1.0)
    out[...] = s / cc


@jax.jit
def _tc_combine(partials, counts):
    pspec = pl.BlockSpec((NC, BN, D_FEAT), lambda i: (0, i, 0))
    return pl.pallas_call(
        _combine_body,
        grid=(N_NODES // BN,),
        in_specs=[pspec] * (NG + 1),
        out_specs=pl.BlockSpec((BN, D_FEAT), lambda i: (i, 0)),
        out_shape=jax.ShapeDtypeStruct((N_NODES, D_FEAT), jnp.float32),
    )(*partials, counts)


def kernel(x, edge_index, edge_attr, W1, b1, W2, b2):
    dst = edge_index[1].astype(jnp.int32)
    src = edge_index[0].astype(jnp.int32)
    dst5 = dst.reshape(NG, NW, NCHG, CH)
    src5 = src.reshape(NG, NW, NCHG, CH)
    eat = edge_attr.T

    w1os = jnp.concatenate(
        [W1[:D_FEAT], W1[D_FEAT + D_EDGE:]], axis=0).astype(jnp.bfloat16)
    w1e = W1[D_FEAT:D_FEAT + D_EDGE].astype(jnp.bfloat16)
    b1r = b1.reshape(1, D_HIDDEN)
    w2 = W2.astype(jnp.bfloat16)
    b2r = b2.reshape(1, D_FEAT)

    zeros_nm = jnp.zeros((N_PAD, D_FEAT), jnp.float32)
    ones_ch = jnp.ones((CH, D_FEAT), jnp.float32)

    partials = []
    for g in range(NG):
        g4 = _sc_gather(x, dst5[g], src5[g])
        msg = _tc_mlp(g4.reshape(EG, D_PAIR),
                      eat[:, g * EG:(g + 1) * EG],
                      w1os, w1e, b1r, w2, b2r)
        partials.append(
            _sc_scatter(msg.reshape(NW, NCHG, CH, D_FEAT), dst5[g], zeros_nm))

    counts = _sc_count(dst.reshape(NW, NCHF, CH), zeros_nm, ones_ch)
    return _tc_combine(partials, counts)


# speedup vs baseline: 4.0794x; 1.0723x over previous
import functools

import jax
import jax.numpy as jnp
from jax import lax
from jax.experimental import pallas as pl
from jax.experimental.pallas import tpu as pltpu
from jax.experimental.pallas import tpu_sc as plsc

N_NODES = 10000
N_EDGES = 320000
D_FEAT = 128
D_PAIR = 2 * D_FEAT
D_EDGE = 16
D_HIDDEN = 512

NC = 2
NS = 16
NW = NC * NS
NG = 5
EG = N_EDGES // NG
EPWG = EG // NW
CH = 80
NCHG = EPWG // CH
NCH2G = (NCHG - 1) // 2
NCHF = N_EDGES // NW // CH
N_PAD = 10240
NPT = N_PAD // NS

_sc_mesh = plsc.VectorSubcoreMesh(core_axis_name="c", subcore_axis_name="s")


def _gather_body(x_hbm, dst_hbm, src_hbm, g_hbm, idx_d, idx_s,
                 bA_d, bA_s, bB_d, bB_s,
                 s_gAd, s_gAs, s_gBd, s_gBs, s_wAd, s_wAs, s_wBd, s_wBs):
    cid = lax.axis_index("c")
    sid = lax.axis_index("s")
    wid = sid * NC + cid
    pltpu.sync_copy(dst_hbm.at[wid], idx_d)
    pltpu.sync_copy(src_hbm.at[wid], idx_s)

    obj_lanes = pl.ds(0, D_FEAT)
    sub_lanes = pl.ds(D_FEAT, D_FEAT)

    pltpu.async_copy(x_hbm.at[idx_d.at[0]], bA_d, s_gAd)
    pltpu.async_copy(x_hbm.at[idx_s.at[0]], bA_s, s_gAs)

    def body(jj, carry):
        j0 = 2 * jj
        j1 = j0 + 1
        j2 = j0 + 2
        pltpu.make_async_copy(x_hbm.at[idx_d.at[j0]], bA_d, s_gAd).wait()
        pltpu.make_async_copy(x_hbm.at[idx_s.at[j0]], bA_s, s_gAs).wait()
        pltpu.async_copy(x_hbm.at[idx_d.at[j1]], bB_d, s_gBd)
        pltpu.async_copy(x_hbm.at[idx_s.at[j1]], bB_s, s_gBs)
        wAd = pltpu.async_copy(bA_d, g_hbm.at[wid, j0, :, obj_lanes], s_wAd)
        wAs = pltpu.async_copy(bA_s, g_hbm.at[wid, j0, :, sub_lanes], s_wAs)
        pltpu.make_async_copy(x_hbm.at[idx_d.at[j1]], bB_d, s_gBd).wait()
        pltpu.make_async_copy(x_hbm.at[idx_s.at[j1]], bB_s, s_gBs).wait()
        wBd = pltpu.async_copy(bB_d, g_hbm.at[wid, j1, :, obj_lanes], s_wBd)
        wBs = pltpu.async_copy(bB_s, g_hbm.at[wid, j1, :, sub_lanes], s_wBs)
        wAd.wait()
        wAs.wait()
        pltpu.async_copy(x_hbm.at[idx_d.at[j2]], bA_d, s_gAd)
        pltpu.async_copy(x_hbm.at[idx_s.at[j2]], bA_s, s_gAs)
        wBd.wait()
        wBs.wait()
        return carry

    lax.fori_loop(0, NCH2G, body, 0)
    j = NCHG - 1
    pltpu.make_async_copy(x_hbm.at[idx_d.at[j]], bA_d, s_gAd).wait()
    pltpu.make_async_copy(x_hbm.at[idx_s.at[j]], bA_s, s_gAs).wait()
    pltpu.sync_copy(bA_d, g_hbm.at[wid, j, :, obj_lanes])
    pltpu.sync_copy(bA_s, g_hbm.at[wid, j, :, sub_lanes])


@jax.jit
def _sc_gather(x, dstg, srcg):
    buf = lambda: pltpu.VMEM((CH, D_FEAT), jnp.float32)
    return pl.kernel(
        _gather_body,
        out_type=jax.ShapeDtypeStruct((NW, NCHG, CH, D_PAIR), jnp.float32),
        mesh=_sc_mesh,
        scratch_types=[
            pltpu.VMEM((NCHG, CH), jnp.int32),
            pltpu.VMEM((NCHG, CH), jnp.int32),
            buf(), buf(), buf(), buf(),
        ] + [pltpu.SemaphoreType.DMA] * 8,
    )(x, dstg, srcg)


def _scatter_body(msg_hbm, dst_hbm, zero_hbm, out_hbm,
                  idx_d, bA, bB, acc, s_lA, s_lB, s_sA, s_sB):
    cid = lax.axis_index("c")
    sid = lax.axis_index("s")
    wid = sid * NC + cid
    pltpu.sync_copy(zero_hbm.at[pl.ds(sid * NPT, NPT)],
                    acc.at[pl.ds(sid * NPT, NPT)])
    pltpu.sync_copy(dst_hbm.at[wid], idx_d)
    plsc.subcore_barrier()

    pltpu.async_copy(msg_hbm.at[wid, 0], bA, s_lA)

    def body(jj, carry):
        j0 = 2 * jj
        j1 = j0 + 1
        j2 = j0 + 2
        pltpu.make_async_copy(msg_hbm.at[wid, j0], bA, s_lA).wait()
        pltpu.async_copy(msg_hbm.at[wid, j1], bB, s_lB)
        sA = pltpu.async_copy(bA, acc.at[idx_d.at[j0]], s_sA, add=True)
        pltpu.make_async_copy(msg_hbm.at[wid, j1], bB, s_lB).wait()
        sB = pltpu.async_copy(bB, acc.at[idx_d.at[j1]], s_sB, add=True)
        sA.wait()
        pltpu.async_copy(msg_hbm.at[wid, j2], bA, s_lA)
        sB.wait()
        return carry

    lax.fori_loop(0, NCH2G, body, 0)
    j = NCHG - 1
    pltpu.make_async_copy(msg_hbm.at[wid, j], bA, s_lA).wait()
    pltpu.sync_copy(bA, acc.at[idx_d.at[j]], add=True)
    plsc.subcore_barrier()
    pltpu.sync_copy(acc.at[pl.ds(sid * NPT, NPT)],
                    out_hbm.at[cid, pl.ds(sid * NPT, NPT)])


@jax.jit
def _sc_scatter(msg4, dstg, zeros_nm):
    return pl.kernel(
        _scatter_body,
        out_type=jax.ShapeDtypeStruct((NC, N_PAD, D_FEAT), jnp.float32),
        mesh=_sc_mesh,
        scratch_types=[
            pltpu.VMEM((NCHG, CH), jnp.int32),
            pltpu.VMEM((CH, D_FEAT), jnp.float32),
            pltpu.VMEM((CH, D_FEAT), jnp.float32),
            pltpu.VMEM_SHARED((N_PAD, D_FEAT), jnp.float32),
        ] + [pltpu.SemaphoreType.DMA] * 4,
    )(msg4, dstg, zeros_nm)


def _count_body(dst_hbm, out_hbm, idx_d, cnt8, tmp16, crow, sh):
    cid = lax.axis_index("c")
    sid = lax.axis_index("s")
    wid = sid * NC + cid
    pltpu.sync_copy(dst_hbm.at[wid], idx_d)

    z16 = jnp.zeros((16,), jnp.float32)

    def zb(k, carry):
        for r in range(8):
            cnt8[r, pl.ds(k * 16, 16)] = z16
        return carry

    lax.fori_loop(0, N_PAD // 16, zb, 0)

    lanes = lax.broadcasted_iota(jnp.int32, (16,), 0)
    rows8 = jnp.bitwise_and(lanes, 7)
    mlo = lanes < 8
    mhi = jnp.logical_not(mlo)
    ones16 = jnp.ones((16,), jnp.float32)

    def hb(j, carry):
        for kk in range(CH // 16):
            iv = idx_d[j, pl.ds(kk * 16, 16)]
            plsc.addupdate_scatter(cnt8, [rows8, iv], ones16, mask=mlo)
            plsc.addupdate_scatter(cnt8, [rows8, iv], ones16, mask=mhi)
        return carry

    lax.fori_loop(0, NCHF, hb, 0)

    def rb(k, carry):
        s = cnt8[0, pl.ds(k * 16, 16)]
        for r in range(1, 8):
            s = s + cnt8[r, pl.ds(k * 16, 16)]
        cnt8[0, pl.ds(k * 16, 16)] = s
        return carry

    lax.fori_loop(0, N_PAD // 16, rb, 0)
    pltpu.sync_copy(cnt8.at[0], sh.at[sid])
    plsc.subcore_barrier()
    pltpu.sync_copy(sh.at[:, pl.ds(sid * NPT, NPT)], tmp16)

    def cb(k, carry):
        s = tmp16[0, pl.ds(k * 16, 16)]
        for r in range(1, NS):
            s = s + tmp16[r, pl.ds(k * 16, 16)]
        crow[pl.ds(k * 16, 16)] = s
        return carry

    lax.fori_loop(0, NPT // 16, cb, 0)
    pltpu.sync_copy(crow, out_hbm.at[cid, pl.ds(sid * NPT, NPT)])


@jax.jit
def _sc_count(dst3):
    return pl.kernel(
        _count_body,
        out_type=jax.ShapeDtypeStruct((NC, N_PAD), jnp.float32),
        mesh=_sc_mesh,
        compiler_params=pltpu.CompilerParams(needs_layout_passes=False),
        scratch_types=[
            pltpu.VMEM((NCHF, CH), jnp.int32),
            pltpu.VMEM((8, N_PAD), jnp.float32),
            pltpu.VMEM((NS, NPT), jnp.float32),
            pltpu.VMEM((NPT,), jnp.float32),
            pltpu.VMEM_SHARED((NS, N_PAD), jnp.float32),
        ],
    )(dst3)


BE = 1280
assert EG % BE == 0


def _mlp_body(g, eat, w1os, w1e, b1, w2, b2, out):
    gb = g[...].astype(jnp.bfloat16)
    e = eat[...].astype(jnp.bfloat16)
    h = jnp.dot(gb, w1os[...], preferred_element_type=jnp.float32)
    h = h + lax.dot_general(e, w1e[...], (((0,), (0,)), ((), ())),
                            preferred_element_type=jnp.float32)
    h = jnp.maximum(h + b1[...], 0.0).astype(jnp.bfloat16)
    out[...] = jnp.dot(h, w2[...], preferred_element_type=jnp.float32) + b2[...]


@jax.jit
def _tc_mlp(g, eat, w1os, w1e, b1, w2, b2):
    grid = (EG // BE,)
    full = lambda shape: pl.BlockSpec(shape, lambda i: (0, 0))
    return pl.pallas_call(
        _mlp_body,
        grid=grid,
        in_specs=[
            pl.BlockSpec((BE, D_PAIR), lambda i: (i, 0)),
            pl.BlockSpec((D_EDGE, BE), lambda i: (0, i)),
            full((D_PAIR, D_HIDDEN)),
            full((D_EDGE, D_HIDDEN)),
            full((1, D_HIDDEN)),
            full((D_HIDDEN, D_FEAT)),
            full((1, D_FEAT)),
        ],
        out_specs=pl.BlockSpec((BE, D_FEAT), lambda i: (i, 0)),
        out_shape=jax.ShapeDtypeStruct((EG, D_FEAT), jnp.float32),
    )(g, eat, w1os, w1e, b1, w2, b2)


BN = 2048


def _combine_body(p0, p1, p2, p3, p4, c, out):
    s = p0[0] + p0[1]
    for p in (p1, p2, p3, p4):
        s = s + p[0] + p[1]
    i = pl.program_id(0)
    cblk = c[:, pl.ds(i * BN, BN)]
    ct = jnp.transpose(cblk)
    cc = jnp.maximum(ct[:, 0:1] + ct[:, 1:2], 1.0)
    out[...] = s / cc


@jax.jit
def _tc_combine(partials, counts):
    pspec = pl.BlockSpec((NC, BN, D_FEAT), lambda i: (0, i, 0))
    cspec = pl.BlockSpec((NC, N_PAD), lambda i: (0, 0))
    return pl.pallas_call(
        _combine_body,
        grid=(pl.cdiv(N_NODES, BN),),
        in_specs=[pspec] * NG + [cspec],
        out_specs=pl.BlockSpec((BN, D_FEAT), lambda i: (i, 0)),
        out_shape=jax.ShapeDtypeStruct((N_NODES, D_FEAT), jnp.float32),
    )(*partials, counts)


def kernel(x, edge_index, edge_attr, W1, b1, W2, b2):
    dst = edge_index[1].astype(jnp.int32)
    src = edge_index[0].astype(jnp.int32)
    dst5 = dst.reshape(NG, NW, NCHG, CH)
    src5 = src.reshape(NG, NW, NCHG, CH)
    eat = edge_attr.T

    w1os = jnp.concatenate(
        [W1[:D_FEAT], W1[D_FEAT + D_EDGE:]], axis=0).astype(jnp.bfloat16)
    w1e = W1[D_FEAT:D_FEAT + D_EDGE].astype(jnp.bfloat16)
    b1r = b1.reshape(1, D_HIDDEN)
    w2 = W2.astype(jnp.bfloat16)
    b2r = b2.reshape(1, D_FEAT)

    zeros_nm = jnp.zeros((N_PAD, D_FEAT), jnp.float32)

    partials = []
    for g in range(NG):
        g4 = _sc_gather(x, dst5[g], src5[g])
        msg = _tc_mlp(g4.reshape(EG, D_PAIR),
                      eat[:, g * EG:(g + 1) * EG],
                      w1os, w1e, b1r, w2, b2r)
        partials.append(
            _sc_scatter(msg.reshape(NW, NCHG, CH, D_FEAT), dst5[g], zeros_nm))

    counts = _sc_count(dst.reshape(NW, NCHF, CH))
    return _tc_combine(partials, counts)
